# baseline (speedup 1.0000x reference)
import functools

import jax
import jax.numpy as jnp
from jax import lax
from jax.experimental import pallas as pl
from jax.experimental.pallas import tpu as pltpu
from jax.experimental.pallas import tpu_sc as plsc

_N = 10000
_E = 320000
_D = 128
_NS = 16
_NP = 10240
_RPT = _NP // _NS
_CH = 160
_EPT = _CH * 128
_EP = _NS * _EPT
_TB = 512

_mesh = plsc.VectorSubcoreMesh(core_axis_name="c", subcore_axis_name="s",
                               num_cores=1)


@functools.partial(
    pl.kernel,
    out_type=jax.ShapeDtypeStruct((2, _NP), jnp.float32),
    mesh=_mesh,
    scratch_types=[
        pltpu.VMEM_SHARED((_NP,), jnp.float32),
        pltpu.VMEM_SHARED((_NP,), jnp.float32),
        pltpu.VMEM((_CH, 2, 128), jnp.int32),
        pltpu.VMEM((_RPT,), jnp.float32),
        pltpu.VMEM((128,), jnp.float32),
        pltpu.SemaphoreType.DMA,
    ],
)
def _deg_kernel(edge_hbm, out_hbm, acc_s, acc_d, pair_v, zb, ones_v, sem):
    sid = lax.axis_index("s")
    pltpu.sync_copy(edge_hbm.at[sid], pair_v)

    def zfill(k, carry):
        zb[pl.ds(k * 16, 16)] = jnp.zeros((16,), jnp.float32)
        return carry

    lax.fori_loop(0, _RPT // 16, zfill, 0)

    def ofill(k, carry):
        ones_v[pl.ds(k * 16, 16)] = jnp.ones((16,), jnp.float32)
        return carry

    lax.fori_loop(0, 8, ofill, 0)

    base = sid * _RPT
    pltpu.sync_copy(zb, acc_s.at[pl.ds(base, _RPT)])
    pltpu.sync_copy(zb, acc_d.at[pl.ds(base, _RPT)])
    plsc.subcore_barrier()

    def fire(j, carry):
        pltpu.async_copy(ones_v, acc_s.at[pair_v.at[j, 0]], sem, add=True)
        pltpu.async_copy(ones_v, acc_d.at[pair_v.at[j, 1]], sem, add=True)
        return carry

    lax.fori_loop(0, _CH, fire, 0)

    def drain(j, carry):
        pltpu.make_async_copy(ones_v, acc_s.at[pair_v.at[j, 0]], sem).wait()
        pltpu.make_async_copy(ones_v, acc_d.at[pair_v.at[j, 1]], sem).wait()
        return carry

    lax.fori_loop(0, _CH, drain, 0)
    plsc.subcore_barrier()
    pltpu.sync_copy(acc_s.at[pl.ds(base, _RPT)], out_hbm.at[0, pl.ds(base, _RPT)])
    pltpu.sync_copy(acc_d.at[pl.ds(base, _RPT)], out_hbm.at[1, pl.ds(base, _RPT)])


@functools.partial(
    pl.kernel,
    out_type=jax.ShapeDtypeStruct((_NP, _D), jnp.float32),
    mesh=_mesh,
    scratch_types=[
        pltpu.VMEM_SHARED((_NP, _D), jnp.float32),
        [pltpu.VMEM((2, 128), jnp.int32) for _ in range(8)],
        [pltpu.VMEM((128, _D), jnp.float32) for _ in range(2)],
        [pltpu.SemaphoreType.DMA for _ in range(8)],
        [pltpu.SemaphoreType.DMA for _ in range(2)],
        [pltpu.SemaphoreType.DMA for _ in range(2)],
    ],
)
def _agg_kernel(g_hbm, edge_hbm, out_hbm, acc, pairs, rows, isems, gsems, ssems):
    sid = lax.axis_index("s")
    start = sid * _CH

    def zfill(k, carry):
        rows[0][k // 8, pl.ds((k % 8) * 16, 16)] = jnp.zeros((16,), jnp.float32)
        return carry

    lax.fori_loop(0, 128 * 8, zfill, 0)

    base = sid * _RPT
    for t in range(_RPT // 128):
        pltpu.sync_copy(rows[0], acc.at[pl.ds(base + t * 128, 128)])
    plsc.subcore_barrier()

    pltpu.sync_copy(edge_hbm.at[start], pairs[0])
    for k in range(1, 6):
        pltpu.async_copy(edge_hbm.at[start + k], pairs[k], isems[k])
    pltpu.async_copy(g_hbm.at[pairs[0].at[0]], rows[0], gsems[0])

    def body(jj, carry):
        for u in range(8):
            i = jj * 8 + u
            p = u % 2
            q = (u + 1) % 2
            s1 = (u + 1) % 8
            s6 = (u + 6) % 8

            @pl.when(i + 1 < _CH)
            def _next_gather():
                pltpu.make_async_copy(edge_hbm.at[start + i + 1], pairs[s1],
                                      isems[s1]).wait()

                @pl.when(i >= 1)
                def _rows_free():
                    pltpu.make_async_copy(rows[q], acc.at[pairs[s1].at[1]],
                                          ssems[q]).wait()

                pltpu.async_copy(g_hbm.at[pairs[s1].at[0]], rows[q], gsems[q])

            pltpu.make_async_copy(g_hbm.at[pairs[u].at[0]], rows[p],
                                  gsems[p]).wait()
            pltpu.async_copy(rows[p], acc.at[pairs[u].at[1]], ssems[p],
                             add=True)

            @pl.when(i + 6 < _CH)
            def _next_pair():
                pltpu.async_copy(edge_hbm.at[start + i + 6], pairs[s6],
                                 isems[s6])
        return carry

    lax.fori_loop(0, _CH // 8, body, 0)
    pltpu.make_async_copy(rows[0], acc.at[pairs[0].at[1]], ssems[0]).wait()
    pltpu.make_async_copy(rows[1], acc.at[pairs[1].at[1]], ssems[1]).wait()
    plsc.subcore_barrier()
    pltpu.sync_copy(acc.at[pl.ds(base, _RPT)], out_hbm.at[pl.ds(base, _RPT)])


def _prescale_body(x_ref, deg_ref, o_ref):
    ds = deg_ref[0]
    o_ref[...] = x_ref[...] * lax.rsqrt(jnp.maximum(ds, 1.0))


def _layer_body(a_ref, deg_ref, w_ref, b_ref, o_ref, *, relu, prescale):
    dd = deg_ref[1]
    h = a_ref[...] * lax.rsqrt(jnp.maximum(dd, 1.0))
    h = jnp.dot(h, w_ref[...], preferred_element_type=jnp.float32) + b_ref[...]
    if relu:
        h = jnp.maximum(h, 0.0)
    if prescale:
        h = h * lax.rsqrt(jnp.maximum(deg_ref[0], 1.0))
    o_ref[...] = h


_deg_spec = pl.BlockSpec((2, _TB, 1), lambda i: (0, i, 0))

_prescale = pl.pallas_call(
    _prescale_body,
    grid=(_NP // _TB,),
    in_specs=[pl.BlockSpec((_TB, _D), lambda i: (i, 0)), _deg_spec],
    out_specs=pl.BlockSpec((_TB, _D), lambda i: (i, 0)),
    out_shape=jax.ShapeDtypeStruct((_NP, _D), jnp.float32),
)


def _make_layer(relu, prescale):
    return pl.pallas_call(
        functools.partial(_layer_body, relu=relu, prescale=prescale),
        grid=(_NP // _TB,),
        in_specs=[
            pl.BlockSpec((_TB, _D), lambda i: (i, 0)),
            _deg_spec,
            pl.BlockSpec((_D, _D), lambda i: (0, 0)),
            pl.BlockSpec((1, _D), lambda i: (0, 0)),
        ],
        out_specs=pl.BlockSpec((_TB, _D), lambda i: (i, 0)),
        out_shape=jax.ShapeDtypeStruct((_NP, _D), jnp.float32),
    )


_layer_mid = _make_layer(relu=True, prescale=True)
_layer_last = _make_layer(relu=False, prescale=False)


def kernel(x, edge_index, W1, b1, W2, b2, W3, b3):
    src = edge_index[0].astype(jnp.int32)
    dst = edge_index[1].astype(jnp.int32)
    pad = _EP - _E
    src = jnp.concatenate([src, jnp.full((pad,), _N, jnp.int32)])
    dst = jnp.concatenate([dst, jnp.full((pad,), _N, jnp.int32)])
    edges = jnp.stack([src.reshape(_NS, _CH, 128),
                       dst.reshape(_NS, _CH, 128)], axis=2)
    edges_f = edges.reshape(_NS * _CH, 2, 128)

    deg = _deg_kernel(edges)
    degr = deg.reshape(2, _NP, 1)

    xp = jnp.pad(x, ((0, _NP - _N), (0, 0)))
    g = _prescale(xp, degr)
    h = _layer_mid(_agg_kernel(g, edges_f), degr, W1, b1.reshape(1, _D))
    h = _layer_mid(_agg_kernel(h, edges_f), degr, W2, b2.reshape(1, _D))
    out = _layer_last(_agg_kernel(h, edges_f), degr, W3, b3.reshape(1, _D))
    return out[:_N]


# speedup vs baseline: 2.6831x; 2.6831x over previous
import functools

import jax
import jax.numpy as jnp
from jax import lax
from jax.experimental import pallas as pl
from jax.experimental.pallas import tpu as pltpu
from jax.experimental.pallas import tpu_sc as plsc

_N = 10000
_E = 320000
_D = 128
_NS = 16
_NP = 10240
_RPT = _NP // _NS
_CH = 160
_EPT = _CH * 128
_EP = _NS * _EPT
_TB = 1024

_mesh = plsc.VectorSubcoreMesh(core_axis_name="c", subcore_axis_name="s")


@functools.partial(
    pl.kernel,
    out_type=jax.ShapeDtypeStruct((2, _NP), jnp.float32),
    mesh=_mesh,
    scratch_types=[
        pltpu.VMEM_SHARED((_NP,), jnp.float32),
        pltpu.VMEM_SHARED((_NP,), jnp.float32),
        pltpu.VMEM((_CH, 2, 128), jnp.int32),
        pltpu.VMEM((_RPT,), jnp.float32),
        pltpu.VMEM((128,), jnp.float32),
        pltpu.SemaphoreType.DMA,
    ],
)
def _deg_kernel(edge_hbm, out_hbm, acc_s, acc_d, pair_v, zb, ones_v, sem):
    cid = lax.axis_index("c")

    @pl.when(cid == 0)
    def _deg_work():
        _deg_body(edge_hbm, out_hbm, acc_s, acc_d, pair_v, zb, ones_v, sem)


def _deg_body(edge_hbm, out_hbm, acc_s, acc_d, pair_v, zb, ones_v, sem):
    sid = lax.axis_index("s")
    pltpu.sync_copy(edge_hbm.at[sid], pair_v)

    def zfill(k, carry):
        zb[pl.ds(k * 16, 16)] = jnp.zeros((16,), jnp.float32)
        return carry

    lax.fori_loop(0, _RPT // 16, zfill, 0)

    def ofill(k, carry):
        ones_v[pl.ds(k * 16, 16)] = jnp.ones((16,), jnp.float32)
        return carry

    lax.fori_loop(0, 8, ofill, 0)

    base = sid * _RPT
    pltpu.sync_copy(zb, acc_s.at[pl.ds(base, _RPT)])
    pltpu.sync_copy(zb, acc_d.at[pl.ds(base, _RPT)])
    plsc.subcore_barrier()

    def fire(j, carry):
        pltpu.async_copy(ones_v, acc_s.at[pair_v.at[j, 0]], sem, add=True)
        pltpu.async_copy(ones_v, acc_d.at[pair_v.at[j, 1]], sem, add=True)
        return carry

    lax.fori_loop(0, _CH, fire, 0)

    def drain(j, carry):
        pltpu.make_async_copy(ones_v, acc_s.at[pair_v.at[j, 0]], sem).wait()
        pltpu.make_async_copy(ones_v, acc_d.at[pair_v.at[j, 1]], sem).wait()
        return carry

    lax.fori_loop(0, _CH, drain, 0)
    plsc.subcore_barrier()
    pltpu.sync_copy(acc_s.at[pl.ds(base, _RPT)], out_hbm.at[0, pl.ds(base, _RPT)])
    pltpu.sync_copy(acc_d.at[pl.ds(base, _RPT)], out_hbm.at[1, pl.ds(base, _RPT)])


@functools.partial(
    pl.kernel,
    out_type=jax.ShapeDtypeStruct((_NP, _D), jnp.float32),
    mesh=_mesh,
    scratch_types=[
        pltpu.VMEM_SHARED((_NP, _D), jnp.float32),
        [pltpu.VMEM((2, 128), jnp.int32) for _ in range(8)],
        [pltpu.VMEM((128, _D), jnp.float32) for _ in range(2)],
        [pltpu.SemaphoreType.DMA for _ in range(8)],
        [pltpu.SemaphoreType.DMA for _ in range(2)],
        [pltpu.SemaphoreType.DMA for _ in range(2)],
    ],
)
def _agg_kernel(g_hbm, edge_hbm, out_hbm, acc, pairs, rows, isems, gsems, ssems):
    cid = lax.axis_index("c")

    @pl.when(cid == 0)
    def _agg_work():
        _agg_body(g_hbm, edge_hbm, out_hbm, acc, pairs, rows, isems, gsems,
                  ssems)


def _agg_body(g_hbm, edge_hbm, out_hbm, acc, pairs, rows, isems, gsems, ssems):
    sid = lax.axis_index("s")
    start = sid * _CH

    def zfill(k, carry):
        rows[0][k // 8, pl.ds((k % 8) * 16, 16)] = jnp.zeros((16,), jnp.float32)
        return carry

    lax.fori_loop(0, 128 * 8, zfill, 0)

    base = sid * _RPT
    for t in range(_RPT // 128):
        pltpu.sync_copy(rows[0], acc.at[pl.ds(base + t * 128, 128)])
    plsc.subcore_barrier()

    pltpu.sync_copy(edge_hbm.at[start], pairs[0])
    for k in range(1, 6):
        pltpu.async_copy(edge_hbm.at[start + k], pairs[k], isems[k])
    pltpu.async_copy(g_hbm.at[pairs[0].at[0]], rows[0], gsems[0])

    def body(jj, carry):
        for u in range(8):
            i = jj * 8 + u
            p = u % 2
            q = (u + 1) % 2
            s1 = (u + 1) % 8
            s6 = (u + 6) % 8

            @pl.when(i + 1 < _CH)
            def _next_gather():
                pltpu.make_async_copy(edge_hbm.at[start + i + 1], pairs[s1],
                                      isems[s1]).wait()

                @pl.when(i >= 1)
                def _rows_free():
                    pltpu.make_async_copy(rows[q], acc.at[pairs[s1].at[1]],
                                          ssems[q]).wait()

                pltpu.async_copy(g_hbm.at[pairs[s1].at[0]], rows[q], gsems[q])

            pltpu.make_async_copy(g_hbm.at[pairs[u].at[0]], rows[p],
                                  gsems[p]).wait()
            pltpu.async_copy(rows[p], acc.at[pairs[u].at[1]], ssems[p],
                             add=True)

            @pl.when(i + 6 < _CH)
            def _next_pair():
                pltpu.async_copy(edge_hbm.at[start + i + 6], pairs[s6],
                                 isems[s6])
        return carry

    lax.fori_loop(0, _CH // 8, body, 0)
    pltpu.make_async_copy(rows[0], acc.at[pairs[0].at[1]], ssems[0]).wait()
    pltpu.make_async_copy(rows[1], acc.at[pairs[1].at[1]], ssems[1]).wait()
    plsc.subcore_barrier()
    pltpu.sync_copy(acc.at[pl.ds(base, _RPT)], out_hbm.at[pl.ds(base, _RPT)])


def _prescale_body(x_ref, deg_ref, o_ref):
    ds = deg_ref[0]
    o_ref[...] = x_ref[...] * lax.rsqrt(jnp.maximum(ds, 1.0))


def _layer_body(a_ref, deg_ref, w_ref, b_ref, o_ref, *, relu, prescale):
    dd = deg_ref[1]
    h = a_ref[...] * lax.rsqrt(jnp.maximum(dd, 1.0))
    h = jnp.dot(h, w_ref[...], preferred_element_type=jnp.float32) + b_ref[...]
    if relu:
        h = jnp.maximum(h, 0.0)
    if prescale:
        h = h * lax.rsqrt(jnp.maximum(deg_ref[0], 1.0))
    o_ref[...] = h


_deg_spec = pl.BlockSpec((2, _TB, 1), lambda i: (0, i, 0))

_prescale = pl.pallas_call(
    _prescale_body,
    grid=(_NP // _TB,),
    in_specs=[pl.BlockSpec((_TB, _D), lambda i: (i, 0)), _deg_spec],
    out_specs=pl.BlockSpec((_TB, _D), lambda i: (i, 0)),
    out_shape=jax.ShapeDtypeStruct((_NP, _D), jnp.float32),
)


def _make_layer(relu, prescale):
    return pl.pallas_call(
        functools.partial(_layer_body, relu=relu, prescale=prescale),
        grid=(_NP // _TB,),
        in_specs=[
            pl.BlockSpec((_TB, _D), lambda i: (i, 0)),
            _deg_spec,
            pl.BlockSpec((_D, _D), lambda i: (0, 0)),
            pl.BlockSpec((1, _D), lambda i: (0, 0)),
        ],
        out_specs=pl.BlockSpec((_TB, _D), lambda i: (i, 0)),
        out_shape=jax.ShapeDtypeStruct((_NP, _D), jnp.float32),
    )


_layer_mid = _make_layer(relu=True, prescale=True)
_layer_last = _make_layer(relu=False, prescale=False)


def kernel(x, edge_index, W1, b1, W2, b2, W3, b3):
    src = edge_index[0].astype(jnp.int32)
    dst = edge_index[1].astype(jnp.int32)
    pad = _EP - _E
    pad_idx = _N + (jnp.arange(pad, dtype=jnp.int32) % (_NP - _N))
    src = jnp.concatenate([src, pad_idx])
    dst = jnp.concatenate([dst, pad_idx])
    edges = jnp.stack([src.reshape(_NS, _CH, 128),
                       dst.reshape(_NS, _CH, 128)], axis=2)
    edges_f = edges.reshape(_NS * _CH, 2, 128)

    deg = _deg_kernel(edges)
    degr = deg.reshape(2, _NP, 1)

    xp = jnp.pad(x, ((0, _NP - _N), (0, 0)))
    g = _prescale(xp, degr)
    h = _layer_mid(_agg_kernel(g, edges_f), degr, W1, b1.reshape(1, _D))
    h = _layer_mid(_agg_kernel(h, edges_f), degr, W2, b2.reshape(1, _D))
    out = _layer_last(_agg_kernel(h, edges_f), degr, W3, b3.reshape(1, _D))
    return out[:_N]


# speedup vs baseline: 2.7097x; 1.0099x over previous
import functools

import jax
import jax.numpy as jnp
from jax import lax
from jax.experimental import pallas as pl
from jax.experimental.pallas import tpu as pltpu
from jax.experimental.pallas import tpu_sc as plsc

_N = 10000
_E = 320000
_D = 128
_NS = 16
_NP = 10240
_RPT = _NP // _NS
_CH = 160
_EPT = _CH * 128
_EP = _NS * _EPT
_TB = 2048

_mesh = plsc.VectorSubcoreMesh(core_axis_name="c", subcore_axis_name="s")


@functools.partial(
    pl.kernel,
    out_type=jax.ShapeDtypeStruct((2, _NP), jnp.float32),
    mesh=_mesh,
    scratch_types=[
        pltpu.VMEM_SHARED((_NP,), jnp.float32),
        pltpu.VMEM_SHARED((_NP,), jnp.float32),
        pltpu.VMEM((_CH, 2, 128), jnp.int32),
        pltpu.VMEM((_RPT,), jnp.float32),
        pltpu.VMEM((128,), jnp.float32),
        pltpu.SemaphoreType.DMA,
    ],
)
def _deg_kernel(edge_hbm, out_hbm, acc_s, acc_d, pair_v, zb, ones_v, sem):
    cid = lax.axis_index("c")

    @pl.when(cid == 0)
    def _deg_work():
        _deg_body(edge_hbm, out_hbm, acc_s, acc_d, pair_v, zb, ones_v, sem)


def _deg_body(edge_hbm, out_hbm, acc_s, acc_d, pair_v, zb, ones_v, sem):
    sid = lax.axis_index("s")
    pltpu.sync_copy(edge_hbm.at[sid], pair_v)

    def zfill(k, carry):
        zb[pl.ds(k * 16, 16)] = jnp.zeros((16,), jnp.float32)
        return carry

    lax.fori_loop(0, _RPT // 16, zfill, 0)

    def ofill(k, carry):
        ones_v[pl.ds(k * 16, 16)] = jnp.ones((16,), jnp.float32)
        return carry

    lax.fori_loop(0, 8, ofill, 0)

    base = sid * _RPT
    pltpu.sync_copy(zb, acc_s.at[pl.ds(base, _RPT)])
    pltpu.sync_copy(zb, acc_d.at[pl.ds(base, _RPT)])
    plsc.subcore_barrier()

    def fire(j, carry):
        pltpu.async_copy(ones_v, acc_s.at[pair_v.at[j, 0]], sem, add=True)
        pltpu.async_copy(ones_v, acc_d.at[pair_v.at[j, 1]], sem, add=True)
        return carry

    lax.fori_loop(0, _CH, fire, 0)

    def drain(j, carry):
        pltpu.make_async_copy(ones_v, acc_s.at[pair_v.at[j, 0]], sem).wait()
        pltpu.make_async_copy(ones_v, acc_d.at[pair_v.at[j, 1]], sem).wait()
        return carry

    lax.fori_loop(0, _CH, drain, 0)
    plsc.subcore_barrier()
    pltpu.sync_copy(acc_s.at[pl.ds(base, _RPT)], out_hbm.at[0, pl.ds(base, _RPT)])
    pltpu.sync_copy(acc_d.at[pl.ds(base, _RPT)], out_hbm.at[1, pl.ds(base, _RPT)])


@functools.partial(
    pl.kernel,
    out_type=jax.ShapeDtypeStruct((_NP, _D), jnp.float32),
    mesh=_mesh,
    scratch_types=[
        pltpu.VMEM_SHARED((_NP, _D), jnp.float32),
        [pltpu.VMEM((2, 128), jnp.int32) for _ in range(8)],
        [pltpu.VMEM((128, _D), jnp.float32) for _ in range(2)],
        [pltpu.SemaphoreType.DMA for _ in range(8)],
        [pltpu.SemaphoreType.DMA for _ in range(2)],
        [pltpu.SemaphoreType.DMA for _ in range(2)],
    ],
)
def _agg_kernel(g_hbm, edge_hbm, out_hbm, acc, pairs, rows, isems, gsems, ssems):
    cid = lax.axis_index("c")

    @pl.when(cid == 0)
    def _agg_work():
        _agg_body(g_hbm, edge_hbm, out_hbm, acc, pairs, rows, isems, gsems,
                  ssems)


def _agg_body(g_hbm, edge_hbm, out_hbm, acc, pairs, rows, isems, gsems, ssems):
    sid = lax.axis_index("s")
    start = sid * _CH

    def zfill(k, carry):
        rows[0][k // 8, pl.ds((k % 8) * 16, 16)] = jnp.zeros((16,), jnp.float32)
        return carry

    lax.fori_loop(0, 128 * 8, zfill, 0)

    base = sid * _RPT
    for t in range(_RPT // 128):
        pltpu.sync_copy(rows[0], acc.at[pl.ds(base + t * 128, 128)])
    plsc.subcore_barrier()

    pltpu.sync_copy(edge_hbm.at[start], pairs[0])
    for k in range(1, 6):
        pltpu.async_copy(edge_hbm.at[start + k], pairs[k], isems[k])
    pltpu.async_copy(g_hbm.at[pairs[0].at[0]], rows[0], gsems[0])

    def body(jj, carry):
        for u in range(8):
            i = jj * 8 + u
            p = u % 2
            q = (u + 1) % 2
            s1 = (u + 1) % 8
            s6 = (u + 6) % 8

            @pl.when(i + 1 < _CH)
            def _next_gather():
                pltpu.make_async_copy(edge_hbm.at[start + i + 1], pairs[s1],
                                      isems[s1]).wait()

                @pl.when(i >= 1)
                def _rows_free():
                    pltpu.make_async_copy(rows[q], acc.at[pairs[s1].at[1]],
                                          ssems[q]).wait()

                pltpu.async_copy(g_hbm.at[pairs[s1].at[0]], rows[q], gsems[q])

            pltpu.make_async_copy(g_hbm.at[pairs[u].at[0]], rows[p],
                                  gsems[p]).wait()
            pltpu.async_copy(rows[p], acc.at[pairs[u].at[1]], ssems[p],
                             add=True)

            @pl.when(i + 6 < _CH)
            def _next_pair():
                pltpu.async_copy(edge_hbm.at[start + i + 6], pairs[s6],
                                 isems[s6])
        return carry

    lax.fori_loop(0, _CH // 8, body, 0)
    pltpu.make_async_copy(rows[0], acc.at[pairs[0].at[1]], ssems[0]).wait()
    pltpu.make_async_copy(rows[1], acc.at[pairs[1].at[1]], ssems[1]).wait()
    plsc.subcore_barrier()
    pltpu.sync_copy(acc.at[pl.ds(base, _RPT)], out_hbm.at[pl.ds(base, _RPT)])


def _prescale_body(x_ref, deg_ref, o_ref):
    ds = deg_ref[0]
    o_ref[...] = x_ref[...] * lax.rsqrt(jnp.maximum(ds, 1.0))


def _layer_body(a_ref, deg_ref, w_ref, b_ref, o_ref, *, relu, prescale):
    dd = deg_ref[1]
    h = a_ref[...] * lax.rsqrt(jnp.maximum(dd, 1.0))
    h = jnp.dot(h, w_ref[...], preferred_element_type=jnp.float32) + b_ref[...]
    if relu:
        h = jnp.maximum(h, 0.0)
    if prescale:
        h = h * lax.rsqrt(jnp.maximum(deg_ref[0], 1.0))
    o_ref[...] = h


_deg_spec = pl.BlockSpec((2, _TB, 1), lambda i: (0, i, 0))

_prescale = pl.pallas_call(
    _prescale_body,
    grid=(_NP // _TB,),
    in_specs=[pl.BlockSpec((_TB, _D), lambda i: (i, 0)), _deg_spec],
    out_specs=pl.BlockSpec((_TB, _D), lambda i: (i, 0)),
    out_shape=jax.ShapeDtypeStruct((_NP, _D), jnp.float32),
)


def _make_layer(relu, prescale):
    return pl.pallas_call(
        functools.partial(_layer_body, relu=relu, prescale=prescale),
        grid=(_NP // _TB,),
        in_specs=[
            pl.BlockSpec((_TB, _D), lambda i: (i, 0)),
            _deg_spec,
            pl.BlockSpec((_D, _D), lambda i: (0, 0)),
            pl.BlockSpec((1, _D), lambda i: (0, 0)),
        ],
        out_specs=pl.BlockSpec((_TB, _D), lambda i: (i, 0)),
        out_shape=jax.ShapeDtypeStruct((_NP, _D), jnp.float32),
    )


_layer_mid = _make_layer(relu=True, prescale=True)
_layer_last = _make_layer(relu=False, prescale=False)


def kernel(x, edge_index, W1, b1, W2, b2, W3, b3):
    src = edge_index[0].astype(jnp.int32)
    dst = edge_index[1].astype(jnp.int32)
    pad = _EP - _E
    pad_idx = _N + (jnp.arange(pad, dtype=jnp.int32) % (_NP - _N))
    src = jnp.concatenate([src, pad_idx])
    dst = jnp.concatenate([dst, pad_idx])
    edges = jnp.stack([src.reshape(_NS, _CH, 128),
                       dst.reshape(_NS, _CH, 128)], axis=2)
    edges_f = edges.reshape(_NS * _CH, 2, 128)

    deg = _deg_kernel(edges)
    degr = deg.reshape(2, _NP, 1)

    xp = jnp.pad(x, ((0, _NP - _N), (0, 0)))
    g = _prescale(xp, degr)
    h = _layer_mid(_agg_kernel(g, edges_f), degr, W1, b1.reshape(1, _D))
    h = _layer_mid(_agg_kernel(h, edges_f), degr, W2, b2.reshape(1, _D))
    out = _layer_last(_agg_kernel(h, edges_f), degr, W3, b3.reshape(1, _D))
    return out[:_N]
